# baseline (speedup 1.0000x reference)
import functools

import jax
import jax.numpy as jnp
from jax import lax
from jax.experimental import pallas as pl
from jax.experimental.pallas import tpu as pltpu
from jax.experimental.pallas import tpu_sc as plsc

NUM_GRAPHS = 512
NC = 2
NS = 16
NW = NC * NS
LANES = 16
CHUNK = 32
GPW = NUM_GRAPHS // NW


def _make_sc_kernel(n, npad, emb):
    mesh = plsc.VectorSubcoreMesh(core_axis_name="c", subcore_axis_name="s")
    nseg = emb // LANES

    @functools.partial(
        pl.kernel,
        out_type=jax.ShapeDtypeStruct((NUM_GRAPHS, emb), jnp.float32),
        mesh=mesh,
        scratch_types=[
            pltpu.VMEM((npad,), jnp.int32),
            pltpu.VMEM((CHUNK, emb), jnp.float32),
            pltpu.VMEM((GPW, emb), jnp.float32),
            pltpu.SMEM((GPW + 1,), jnp.int32),
        ],
    )
    def k(hs_hbm, batch_hbm, out_hbm, batch_v, buf_v, res_v, bnd_s):
        cid = lax.axis_index("c")
        sid = lax.axis_index("s")
        wid = cid * NS + sid

        pltpu.sync_copy(batch_hbm, batch_v)

        @pl.loop(0, GPW + 1)
        def _(kk):
            t = wid * GPW + kk

            def bs_body(_, lohi):
                lo, hi = lohi
                mid = lax.div(lo + hi, 2)
                v = batch_v[pl.ds(mid, LANES)][0]
                lo2 = jnp.where(v < t, mid + 1, lo)
                hi2 = jnp.where(v < t, hi, mid)
                return (lo2, hi2)

            lo, _hi = lax.fori_loop(0, 17, bs_body,
                                    (jnp.int32(0), jnp.int32(n)))
            bnd_s[kk] = lo

        zero16 = jnp.zeros((LANES,), jnp.float32)

        @pl.loop(0, GPW)
        def _(g):
            s = bnd_s[g]
            e = bnd_s[g + 1]
            cnt = e - s
            abase = lax.div(s, 8) * 8
            off = s - abase
            span = e - abase
            nch = lax.div(span + (CHUNK - 1), CHUNK)

            def chunk_body(c, accs):
                pltpu.sync_copy(hs_hbm.at[pl.ds(abase + c * CHUNK, CHUNK)],
                                buf_v)
                r0 = jnp.where(c == 0, off, jnp.int32(0))
                m = jnp.minimum(jnp.int32(CHUNK), span - c * CHUNK)
                accs = list(accs)
                for r in range(CHUNK):
                    pred = (r >= r0) & (r < m)
                    for j in range(nseg):
                        v = buf_v[r, pl.ds(j * LANES, LANES)]
                        accs[j] = accs[j] + jnp.where(pred, v, 0.0)
                return tuple(accs)

            accs = lax.fori_loop(0, nch, chunk_body, (zero16,) * nseg)

            denom = jnp.full((LANES,),
                             jnp.maximum(cnt, 1).astype(jnp.float32))
            for j in range(nseg):
                res_v[g, pl.ds(j * LANES, LANES)] = accs[j] / denom

        pltpu.sync_copy(res_v, out_hbm.at[pl.ds(wid * GPW, GPW)])

    return k


def kernel(hs, batch):
    n, emb = hs.shape
    npad = ((n + CHUNK - 1) // CHUNK + 1) * CHUNK
    hs_pad = jnp.concatenate(
        [hs, jnp.zeros((npad - n, emb), hs.dtype)], axis=0)
    batch_pad = jnp.concatenate(
        [batch.astype(jnp.int32),
         jnp.full((npad - n,), NUM_GRAPHS, jnp.int32)], axis=0)
    return _make_sc_kernel(n, npad, emb)(hs_pad, batch_pad)


# speedup vs baseline: 1.2728x; 1.2728x over previous
import functools

import jax
import jax.numpy as jnp
from jax import lax
from jax.experimental import pallas as pl
from jax.experimental.pallas import tpu as pltpu
from jax.experimental.pallas import tpu_sc as plsc

NUM_GRAPHS = 512
NC = 2
NS = 16
NW = NC * NS
LANES = 16
CHUNK = 32
GPW = NUM_GRAPHS // NW


def _make_sc_kernel(n, npad, emb):
    mesh = plsc.VectorSubcoreMesh(core_axis_name="c", subcore_axis_name="s")
    nseg = emb // LANES

    @functools.partial(
        pl.kernel,
        out_type=jax.ShapeDtypeStruct((NUM_GRAPHS, emb), jnp.float32),
        mesh=mesh,
        scratch_types=[
            pltpu.VMEM((npad,), jnp.int32),
            pltpu.VMEM((CHUNK, emb), jnp.float32),
            pltpu.VMEM((GPW, emb), jnp.float32),
            pltpu.SMEM((GPW + 1,), jnp.int32),
        ],
    )
    def k(hs_hbm, batch_hbm, out_hbm, batch_v, buf_v, res_v, bnd_s):
        cid = lax.axis_index("c")
        sid = lax.axis_index("s")
        wid = cid * NS + sid

        pltpu.sync_copy(batch_hbm, batch_v)

        @pl.loop(0, GPW + 1)
        def _(kk):
            t = wid * GPW + kk

            def bs_body(_, lohi):
                lo, hi = lohi
                mid = lax.div(lo + hi, 2)
                v = batch_v[pl.ds(mid, LANES)][0]
                lo2 = jnp.where(v < t, mid + 1, lo)
                hi2 = jnp.where(v < t, hi, mid)
                return (lo2, hi2)

            lo, _hi = lax.fori_loop(0, 17, bs_body,
                                    (jnp.int32(0), jnp.int32(n)))
            bnd_s[kk] = lo

        zero16 = jnp.zeros((LANES,), jnp.float32)

        @pl.loop(0, GPW)
        def _(g):
            s = bnd_s[g]
            e = bnd_s[g + 1]
            cnt = e - s
            abase = lax.div(s, 8) * 8
            off = s - abase
            span = e - abase
            nch = lax.div(span + (CHUNK - 1), CHUNK)

            def chunk_body(c, accs):
                pltpu.sync_copy(hs_hbm.at[pl.ds(abase + c * CHUNK, CHUNK)],
                                buf_v)
                r0 = jnp.where(c == 0, off, jnp.int32(0))
                m = jnp.minimum(jnp.int32(CHUNK), span - c * CHUNK)

                def row_body(r, accs):
                    return tuple(
                        accs[j] + buf_v[r, pl.ds(j * LANES, LANES)]
                        for j in range(nseg))

                return lax.fori_loop(r0, m, row_body, accs)

            accs = lax.fori_loop(0, nch, chunk_body, (zero16,) * nseg)

            denom = jnp.full((LANES,),
                             jnp.maximum(cnt, 1).astype(jnp.float32))
            for j in range(nseg):
                res_v[g, pl.ds(j * LANES, LANES)] = accs[j] / denom

        pltpu.sync_copy(res_v, out_hbm.at[pl.ds(wid * GPW, GPW)])

    return k


def kernel(hs, batch):
    n, emb = hs.shape
    npad = ((n + CHUNK - 1) // CHUNK + 1) * CHUNK
    hs_pad = jnp.concatenate(
        [hs, jnp.zeros((npad - n, emb), hs.dtype)], axis=0)
    batch_pad = jnp.concatenate(
        [batch.astype(jnp.int32),
         jnp.full((npad - n,), NUM_GRAPHS, jnp.int32)], axis=0)
    return _make_sc_kernel(n, npad, emb)(hs_pad, batch_pad)


# speedup vs baseline: 1.7425x; 1.3690x over previous
import functools

import jax
import jax.numpy as jnp
from jax import lax
from jax.experimental import pallas as pl
from jax.experimental.pallas import tpu as pltpu
from jax.experimental.pallas import tpu_sc as plsc

NUM_GRAPHS = 512
NC = 2
NS = 16
NW = NC * NS
LANES = 16
CHUNK = 32
GPW = NUM_GRAPHS // NW


def _make_sc_kernel(n, npad, emb):
    mesh = plsc.VectorSubcoreMesh(core_axis_name="c", subcore_axis_name="s")
    nseg = emb // LANES

    @functools.partial(
        pl.kernel,
        out_type=jax.ShapeDtypeStruct((NUM_GRAPHS, emb), jnp.float32),
        mesh=mesh,
        scratch_types=[
            pltpu.VMEM((npad,), jnp.int32),
            pltpu.VMEM((CHUNK, emb), jnp.float32),
            pltpu.VMEM((GPW, emb), jnp.float32),
            pltpu.SMEM((GPW + 1,), jnp.int32),
        ],
    )
    def k(hs_hbm, batch_hbm, out_hbm, batch_v, buf_v, res_v, bnd_s):
        cid = lax.axis_index("c")
        sid = lax.axis_index("s")
        wid = cid * NS + sid

        pltpu.sync_copy(batch_hbm, batch_v)

        @pl.loop(0, GPW + 1)
        def _(kk):
            t = wid * GPW + kk

            def bs_body(_, lohi):
                lo, hi = lohi
                mid = lax.div(lo + hi, 2)
                v = batch_v[pl.ds(mid, LANES)][0]
                lo2 = jnp.where(v < t, mid + 1, lo)
                hi2 = jnp.where(v < t, hi, mid)
                return (lo2, hi2)

            lo, _hi = lax.fori_loop(0, 17, bs_body,
                                    (jnp.int32(0), jnp.int32(n)))
            bnd_s[kk] = lo

        zero16 = jnp.zeros((LANES,), jnp.float32)

        @pl.loop(0, GPW)
        def _(g):
            s = bnd_s[g]
            e = bnd_s[g + 1]
            cnt = e - s
            abase = lax.div(s, 8) * 8
            span = e - abase
            nch = lax.div(span + (CHUNK - 1), CHUNK)

            def chunk_body(c, accs):
                g0 = abase + c * CHUNK
                b = jnp.minimum(g0, jnp.int32(n - CHUNK))
                pltpu.sync_copy(hs_hbm.at[pl.ds(b, CHUNK)], buf_v)
                r0 = jnp.maximum(g0, s) - b
                m = jnp.minimum(g0 + CHUNK, e) - b

                def row_body(r, accs):
                    return tuple(
                        accs[j] + buf_v[r, pl.ds(j * LANES, LANES)]
                        for j in range(nseg))

                return lax.fori_loop(r0, m, row_body, accs)

            accs = lax.fori_loop(0, nch, chunk_body, (zero16,) * nseg)

            denom = jnp.full((LANES,),
                             jnp.maximum(cnt, 1).astype(jnp.float32))
            for j in range(nseg):
                res_v[g, pl.ds(j * LANES, LANES)] = accs[j] / denom

        pltpu.sync_copy(res_v, out_hbm.at[pl.ds(wid * GPW, GPW)])

    return k


def kernel(hs, batch):
    n, emb = hs.shape
    if n % 8:
        pad8 = 8 - n % 8
        hs = jnp.concatenate([hs, jnp.zeros((pad8, emb), hs.dtype)], axis=0)
        batch = jnp.concatenate(
            [batch.astype(jnp.int32),
             jnp.full((pad8,), NUM_GRAPHS, jnp.int32)])
        n += pad8
    npad = ((n + CHUNK - 1) // CHUNK + 1) * CHUNK
    batch_pad = jnp.concatenate(
        [batch.astype(jnp.int32),
         jnp.full((npad - n,), NUM_GRAPHS, jnp.int32)], axis=0)
    return _make_sc_kernel(n, npad, emb)(hs, batch_pad)


# speedup vs baseline: 2.0682x; 1.1869x over previous
import functools

import jax
import jax.numpy as jnp
from jax import lax
from jax.experimental import pallas as pl
from jax.experimental.pallas import tpu as pltpu
from jax.experimental.pallas import tpu_sc as plsc

NUM_GRAPHS = 512
NC = 2
NS = 16
NW = NC * NS
LANES = 16
CHUNK = 32
GPW = NUM_GRAPHS // NW


def _make_sc_kernel(n, npad, emb):
    mesh = plsc.VectorSubcoreMesh(core_axis_name="c", subcore_axis_name="s")
    nseg = emb // LANES

    @functools.partial(
        pl.kernel,
        out_type=jax.ShapeDtypeStruct((NUM_GRAPHS, emb), jnp.float32),
        mesh=mesh,
        scratch_types=[
            pltpu.VMEM((npad,), jnp.int32),
            pltpu.VMEM((CHUNK, emb), jnp.float32),
            pltpu.VMEM((CHUNK, emb), jnp.float32),
            pltpu.VMEM((GPW, emb), jnp.float32),
            pltpu.SMEM((GPW + 1,), jnp.int32),
            pltpu.SemaphoreType.DMA,
            pltpu.SemaphoreType.DMA,
        ],
    )
    def k(hs_hbm, batch_hbm, out_hbm,
          batch_v, buf0_v, buf1_v, res_v, bnd_s, sem0, sem1):
        cid = lax.axis_index("c")
        sid = lax.axis_index("s")
        wid = cid * NS + sid

        pltpu.sync_copy(batch_hbm, batch_v)

        @pl.loop(0, GPW + 1)
        def _(kk):
            t = wid * GPW + kk

            def bs_body(_, lohi):
                lo, hi = lohi
                mid = lax.div(lo + hi, 2)
                v = batch_v[pl.ds(mid, LANES)][0]
                lo2 = jnp.where(v < t, mid + 1, lo)
                hi2 = jnp.where(v < t, hi, mid)
                return (lo2, hi2)

            lo, _hi = lax.fori_loop(0, 17, bs_body,
                                    (jnp.int32(0), jnp.int32(n)))
            bnd_s[kk] = lo

        zero16 = jnp.zeros((LANES,), jnp.float32)

        @pl.loop(0, GPW)
        def _(g):
            s = bnd_s[g]
            e = bnd_s[g + 1]
            cnt = e - s
            abase = lax.div(s, 8) * 8
            span = e - abase
            nch = lax.div(span + (CHUNK - 1), CHUNK)

            def chunk_base(c):
                return jnp.minimum(abase + c * CHUNK,
                                   jnp.int32(n - CHUNK))

            def start(c, buf, sem):
                pltpu.async_copy(hs_hbm.at[pl.ds(chunk_base(c), CHUNK)],
                                 buf, sem)

            def wait(buf, sem):
                pltpu.make_async_copy(hs_hbm.at[pl.ds(0, CHUNK)],
                                      buf, sem).wait()

            def accum(c, buf, accs):
                g0 = abase + c * CHUNK
                b = chunk_base(c)
                r0 = jnp.maximum(g0, s) - b
                m = jnp.minimum(g0 + CHUNK, e) - b

                def row_body(r, accs):
                    return tuple(
                        accs[j] + buf[r, pl.ds(j * LANES, LANES)]
                        for j in range(nseg))

                return lax.fori_loop(r0, m, row_body, accs)

            @pl.when(nch > 0)
            def _():
                start(0, buf0_v, sem0)

            def pair_body(p, accs):
                c0 = 2 * p
                c1 = c0 + 1
                wait(buf0_v, sem0)

                @pl.when(c1 < nch)
                def _():
                    start(c1, buf1_v, sem1)

                accs = accum(c0, buf0_v, accs)

                @pl.when(c1 < nch)
                def _():
                    wait(buf1_v, sem1)

                    @pl.when(c1 + 1 < nch)
                    def _():
                        start(c1 + 1, buf0_v, sem0)

                accs = accum(c1, buf1_v, accs)
                return accs

            npairs = lax.div(nch + 1, 2)
            accs = lax.fori_loop(0, npairs, pair_body, (zero16,) * nseg)

            denom = jnp.full((LANES,),
                             jnp.maximum(cnt, 1).astype(jnp.float32))
            for j in range(nseg):
                res_v[g, pl.ds(j * LANES, LANES)] = accs[j] / denom

        pltpu.sync_copy(res_v, out_hbm.at[pl.ds(wid * GPW, GPW)])

    return k


def kernel(hs, batch):
    n, emb = hs.shape
    if n % 8:
        pad8 = 8 - n % 8
        hs = jnp.concatenate([hs, jnp.zeros((pad8, emb), hs.dtype)], axis=0)
        batch = jnp.concatenate(
            [batch.astype(jnp.int32),
             jnp.full((pad8,), NUM_GRAPHS, jnp.int32)])
        n += pad8
    npad = ((n + CHUNK - 1) // CHUNK + 1) * CHUNK
    batch_pad = jnp.concatenate(
        [batch.astype(jnp.int32),
         jnp.full((npad - n,), NUM_GRAPHS, jnp.int32)], axis=0)
    return _make_sc_kernel(n, npad, emb)(hs, batch_pad)
